# baseline (speedup 1.0000x reference)
import functools

import jax
import jax.numpy as jnp
from jax import lax
from jax.experimental import pallas as pl
from jax.experimental.pallas import tpu as pltpu
from jax.experimental.pallas import tpu_sc as plsc

LANES = 16
NUM_CORES = 2
NUM_SUBCORES = 16
NUM_WORKERS = NUM_CORES * NUM_SUBCORES

_MESH = dict(core_axis_name="c", subcore_axis_name="s", num_cores=NUM_CORES)


def _make_transpose_kernel(voc, dim):
    nblk_full = voc // 128
    iters = 246
    mesh = plsc.VectorSubcoreMesh(**_MESH)

    @functools.partial(
        pl.kernel,
        out_type=jax.ShapeDtypeStruct((voc // 2, 2 * dim), jnp.float32),
        mesh=mesh,
        scratch_types=[
            pltpu.VMEM((dim, 128), jnp.float32),
            pltpu.VMEM((dim, 128), jnp.float32),
            pltpu.VMEM((64, 2 * dim), jnp.float32),
            pltpu.VMEM((64, 2 * dim), jnp.float32),
            pltpu.SemaphoreType.DMA,
            pltpu.SemaphoreType.DMA,
            pltpu.SemaphoreType.DMA,
            pltpu.SemaphoreType.DMA,
        ],
        compiler_params=pltpu.CompilerParams(use_tc_tiling_on_sc=True, needs_layout_passes=False),
    )
    def transpose_kernel(embt_hbm, pairs_hbm, in0, in1, out0, out1,
                         sem_i0, sem_i1, sem_s0, sem_s1):
        wid = lax.axis_index("s") * NUM_CORES + lax.axis_index("c")
        in_bufs, out_bufs = (in0, in1), (out0, out1)
        sem_i, sem_s = (sem_i0, sem_i1), (sem_s0, sem_s1)
        iotav = [jnp.arange(16, dtype=jnp.int32) + 16 * q for q in range(dim // 16)]

        def blk_id(k):
            return jnp.minimum(wid + NUM_WORKERS * k, nblk_full - 1)

        def in_src(k):
            return embt_hbm.at[pl.ds(0, dim), pl.ds(blk_id(k) * 128, 128)]

        def out_dst(k):
            return pairs_hbm.at[pl.ds(blk_id(k) * 64, 64)]

        def transpose_block(src, dst):
            def row_body(r, carry):
                s0 = jnp.full((16,), 2 * r, dtype=jnp.int32)
                s1 = s0 + 1
                for g in range(2 * dim // 16):
                    col = s0 if g < dim // 16 else s1
                    v = plsc.load_gather(src, [iotav[g % (dim // 16)], col])
                    dst[r, pl.ds(16 * g, 16)] = v
                return carry

            lax.fori_loop(0, 64, row_body, 0)

        pltpu.async_copy(in_src(0), in0, sem_i0)

        def step(k, a):
            b = 1 - a

            @pl.when(k >= 2)
            def _():
                pltpu.make_async_copy(out_bufs[a], out_dst(k - 2), sem_s[a]).wait()

            pltpu.make_async_copy(in_src(k), in_bufs[a], sem_i[a]).wait()
            pltpu.async_copy(in_src(k + 1), in_bufs[b], sem_i[b])
            transpose_block(in_bufs[a], out_bufs[a])
            pltpu.async_copy(out_bufs[a], out_dst(k), sem_s[a])

        def pair_body(k2, carry):
            step(2 * k2, 0)
            step(2 * k2 + 1, 1)
            return carry

        lax.fori_loop(0, iters // 2, pair_body, 0)
        pltpu.make_async_copy(in_src(iters), in_bufs[0], sem_i[0]).wait()
        pltpu.make_async_copy(out_bufs[0], out_dst(iters - 2), sem_s[0]).wait()
        pltpu.make_async_copy(out_bufs[1], out_dst(iters - 1), sem_s[1]).wait()

    return transpose_kernel


def _make_gather_kernel(batch, seq, voc, dim):
    b_per_w = batch // NUM_WORKERS
    mesh = plsc.VectorSubcoreMesh(**_MESH)

    @functools.partial(
        pl.kernel,
        out_type=jax.ShapeDtypeStruct((seq, dim, batch), jnp.float32),
        mesh=mesh,
        scratch_types=[
            pltpu.VMEM((seq, b_per_w), jnp.int32),
            pltpu.VMEM((dim, 512), jnp.float32),
            pltpu.VMEM((b_per_w,), jnp.int32),
            pltpu.VMEM((b_per_w,), jnp.int32),
            pltpu.VMEM((b_per_w, 2 * dim), jnp.float32),
            pltpu.VMEM((b_per_w, 2 * dim), jnp.float32),
            pltpu.VMEM((dim, b_per_w), jnp.float32),
            pltpu.VMEM((dim, b_per_w), jnp.float32),
            pltpu.SemaphoreType.DMA,
            pltpu.SemaphoreType.DMA,
            pltpu.SemaphoreType.DMA,
            pltpu.SemaphoreType.DMA,
        ],
        compiler_params=pltpu.CompilerParams(use_tc_tiling_on_sc=True, needs_layout_passes=False),
    )
    def gather_kernel(xt_hbm, pairs_hbm, pos_hbm, out_hbm,
                      x_v, pos_v, idx0, idx1, rows0, rows1, ov0, ov1,
                      sem_g0, sem_g1, sem_o0, sem_o1):
        wid = lax.axis_index("s") * NUM_CORES + lax.axis_index("c")
        b0 = wid * b_per_w
        idx_bufs, row_bufs, out_bufs = (idx0, idx1), (rows0, rows1), (ov0, ov1)
        sem_g, sem_o = (sem_g0, sem_g1), (sem_o0, sem_o1)
        ngrp = b_per_w // 16
        iotav = [jnp.arange(16, dtype=jnp.int32) + 16 * q for q in range(ngrp)]

        pltpu.sync_copy(xt_hbm.at[pl.ds(0, seq), pl.ds(b0, b_per_w)], x_v)
        pltpu.sync_copy(pos_hbm, pos_v)

        def prep_idx(s, b):
            hv = []
            for g in range(ngrp):
                xv = x_v[s, pl.ds(16 * g, 16)]
                idx_bufs[b][pl.ds(16 * g, 16)] = lax.shift_right_logical(xv, 1)
                hv.append(jnp.bitwise_and(xv, 1) * dim)
            return tuple(hv)

        def start_gather(b):
            pltpu.async_copy(pairs_hbm.at[idx_bufs[b]], row_bufs[b], sem_g[b])

        def wait_gather(b):
            pltpu.make_async_copy(
                pairs_hbm.at[idx_bufs[b]], row_bufs[b], sem_g[b]
            ).wait()

        def out_dst(s):
            return out_hbm.at[s, pl.ds(0, dim), pl.ds(b0, b_per_w)]

        def compute(s, hv, a):
            sv = jnp.full((16,), s, dtype=jnp.int32)

            def d_body(d, hvc):
                p = plsc.load_gather(pos_v, [jnp.full((16,), d, jnp.int32), sv])
                for g in range(ngrp):
                    v = plsc.load_gather(row_bufs[a], [iotav[g], hvc[g] + d])
                    out_bufs[a][d, pl.ds(16 * g, 16)] = v + p
                return hvc

            lax.fori_loop(0, dim, d_body, tuple(hv))

        hv0 = prep_idx(0, 0)
        start_gather(0)

        def step(s, a, hv_a):
            b = 1 - a

            @pl.when(s >= 2)
            def _():
                pltpu.make_async_copy(out_bufs[a], out_dst(s - 2), sem_o[a]).wait()

            wait_gather(a)
            hv_b = prep_idx(jnp.minimum(s + 1, seq - 1), b)
            start_gather(b)
            compute(s, hv_a, a)
            pltpu.async_copy(out_bufs[a], out_dst(s), sem_o[a])
            return hv_b

        def pair_body(s2, hv_a):
            hv_b = step(2 * s2, 0, hv_a)
            hv_a2 = step(2 * s2 + 1, 1, hv_b)
            return hv_a2

        lax.fori_loop(0, seq // 2, pair_body, tuple(hv0))
        wait_gather(0)
        pltpu.make_async_copy(out_bufs[0], out_dst(seq - 2), sem_o[0]).wait()
        pltpu.make_async_copy(out_bufs[1], out_dst(seq - 1), sem_o[1]).wait()

    return gather_kernel


def kernel(x, emb, posemb):
    batch, seq = x.shape
    voc, dim = emb.shape

    x_t = x.T
    emb_t = emb.T
    pos_t = posemb.T

    pairs = _make_transpose_kernel(voc, dim)(emb_t)
    ntail = (voc % 128) // 2
    if ntail:
        tail = emb[voc - 2 * ntail :, :].reshape(ntail, 2 * dim)
        pairs = lax.dynamic_update_slice(
            pairs, tail, (jnp.int32(voc // 2 - ntail), jnp.int32(0))
        )
    out_t = _make_gather_kernel(batch, seq, voc, dim)(x_t, pairs, pos_t)
    return out_t.transpose(2, 0, 1)


# speedup vs baseline: 1.5293x; 1.5293x over previous
import functools

import jax
import jax.numpy as jnp
from jax import lax
from jax.experimental import pallas as pl
from jax.experimental.pallas import tpu as pltpu
from jax.experimental.pallas import tpu_sc as plsc

LANES = 16
NUM_CORES = 2
NUM_SUBCORES = 16
NUM_WORKERS = NUM_CORES * NUM_SUBCORES

_MESH = dict(core_axis_name="c", subcore_axis_name="s", num_cores=NUM_CORES)


def _make_transpose_kernel(voc, dim):
    nblk_full = voc // 128
    iters = 246
    mesh = plsc.VectorSubcoreMesh(**_MESH)

    @functools.partial(
        pl.kernel,
        out_type=jax.ShapeDtypeStruct((voc // 2, 2 * dim), jnp.float32),
        mesh=mesh,
        scratch_types=[
            pltpu.VMEM((dim, 128), jnp.float32),
            pltpu.VMEM((dim, 128), jnp.float32),
            pltpu.VMEM((64, 2 * dim), jnp.float32),
            pltpu.VMEM((64, 2 * dim), jnp.float32),
            pltpu.SemaphoreType.DMA,
            pltpu.SemaphoreType.DMA,
            pltpu.SemaphoreType.DMA,
            pltpu.SemaphoreType.DMA,
        ],
        compiler_params=pltpu.CompilerParams(use_tc_tiling_on_sc=True, needs_layout_passes=False),
    )
    def transpose_kernel(embt_hbm, pairs_hbm, in0, in1, out0, out1,
                         sem_i0, sem_i1, sem_s0, sem_s1):
        wid = lax.axis_index("s") * NUM_CORES + lax.axis_index("c")
        in_bufs, out_bufs = (in0, in1), (out0, out1)
        sem_i, sem_s = (sem_i0, sem_i1), (sem_s0, sem_s1)
        iotav = [jnp.arange(16, dtype=jnp.int32) + 16 * q for q in range(dim // 16)]

        def blk_id(k):
            return jnp.minimum(wid + NUM_WORKERS * k, nblk_full - 1)

        def in_src(k):
            return embt_hbm.at[pl.ds(0, dim), pl.ds(blk_id(k) * 128, 128)]

        def out_dst(k):
            return pairs_hbm.at[pl.ds(blk_id(k) * 64, 64)]

        iota = jnp.arange(16, dtype=jnp.int32)
        cvs = [(iota + j) % 16 for j in range(16)]

        def transpose_block(src, dst):
            def blk_body(t, carry):
                c0 = t * 16
                for q in range(dim // 16):
                    d0v = iotav[q]
                    for j in range(16):
                        c = cvs[j] + c0
                        r = lax.shift_right_logical(c, 1)
                        col = jnp.bitwise_and(c, 1) * dim + d0v
                        v = plsc.load_gather(src, [d0v, c])
                        plsc.store_scatter(dst, [r, col], v)
                return carry

            lax.fori_loop(0, 8, blk_body, 0)

        pltpu.async_copy(in_src(0), in0, sem_i0)

        def step(k, a):
            b = 1 - a

            @pl.when(k >= 2)
            def _():
                pltpu.make_async_copy(out_bufs[a], out_dst(k - 2), sem_s[a]).wait()

            pltpu.make_async_copy(in_src(k), in_bufs[a], sem_i[a]).wait()
            pltpu.async_copy(in_src(k + 1), in_bufs[b], sem_i[b])
            transpose_block(in_bufs[a], out_bufs[a])
            pltpu.async_copy(out_bufs[a], out_dst(k), sem_s[a])

        def pair_body(k2, carry):
            step(2 * k2, 0)
            step(2 * k2 + 1, 1)
            return carry

        lax.fori_loop(0, iters // 2, pair_body, 0)
        pltpu.make_async_copy(in_src(iters), in_bufs[0], sem_i[0]).wait()
        pltpu.make_async_copy(out_bufs[0], out_dst(iters - 2), sem_s[0]).wait()
        pltpu.make_async_copy(out_bufs[1], out_dst(iters - 1), sem_s[1]).wait()

    return transpose_kernel


def _make_gather_kernel(batch, seq, voc, dim):
    b_per_w = batch // NUM_WORKERS
    mesh = plsc.VectorSubcoreMesh(**_MESH)

    @functools.partial(
        pl.kernel,
        out_type=jax.ShapeDtypeStruct((seq, dim, batch), jnp.float32),
        mesh=mesh,
        scratch_types=[
            pltpu.VMEM((seq, b_per_w), jnp.int32),
            pltpu.VMEM((dim, 256), jnp.float32),
            pltpu.VMEM((b_per_w,), jnp.int32),
            pltpu.VMEM((b_per_w,), jnp.int32),
            pltpu.VMEM((b_per_w,), jnp.int32),
            pltpu.VMEM((b_per_w,), jnp.int32),
            pltpu.VMEM((b_per_w, 2 * dim), jnp.float32),
            pltpu.VMEM((b_per_w, 2 * dim), jnp.float32),
            pltpu.VMEM((dim, b_per_w), jnp.float32),
            pltpu.VMEM((dim, b_per_w), jnp.float32),
            pltpu.SemaphoreType.DMA,
            pltpu.SemaphoreType.DMA,
            pltpu.SemaphoreType.DMA,
            pltpu.SemaphoreType.DMA,
        ],
        compiler_params=pltpu.CompilerParams(use_tc_tiling_on_sc=True, needs_layout_passes=False),
    )
    def gather_kernel(xt_hbm, pairs_hbm, pos_hbm, out_hbm,
                      x_v, pos_v, idx0, idx1, hb0, hb1, rows0, rows1, ov0, ov1,
                      sem_g0, sem_g1, sem_o0, sem_o1):
        wid = lax.axis_index("s") * NUM_CORES + lax.axis_index("c")
        b0 = wid * b_per_w
        idx_bufs, hbufs = (idx0, idx1), (hb0, hb1)
        row_bufs, out_bufs = (rows0, rows1), (ov0, ov1)
        sem_g, sem_o = (sem_g0, sem_g1), (sem_o0, sem_o1)
        ngrp = b_per_w // 16
        iota = jnp.arange(16, dtype=jnp.int32)
        iotav = [iota + 16 * q for q in range(ngrp)]
        cvs = [(iota + j) % 16 for j in range(16)]

        pltpu.sync_copy(xt_hbm.at[pl.ds(0, seq), pl.ds(b0, b_per_w)], x_v)
        pltpu.sync_copy(pos_hbm.at[pl.ds(0, dim), pl.ds(0, 256)], pos_v)

        def prep_idx(s, b):
            for g in range(ngrp):
                sl = pl.ds(16 * g, 16)
                xv = x_v[s, sl]
                idx_bufs[b][sl] = lax.shift_right_logical(xv, 1)
                hbufs[b][sl] = jnp.bitwise_and(xv, 1) * dim

        def start_gather(b):
            pltpu.async_copy(pairs_hbm.at[idx_bufs[b]], row_bufs[b], sem_g[b])

        def wait_gather(b):
            pltpu.make_async_copy(
                pairs_hbm.at[idx_bufs[b]], row_bufs[b], sem_g[b]
            ).wait()

        def out_dst(s):
            return out_hbm.at[s, pl.ds(0, dim), pl.ds(b0, b_per_w)]

        def compute(s, a):
            sv = jnp.full((16,), s, dtype=jnp.int32)
            posq = [
                plsc.load_gather(pos_v, [iotav[q], sv]) for q in range(dim // 16)
            ]

            def b_body(t, carry):
                bb = t * 16
                for q in range(dim // 16):
                    d0v = iotav[q]
                    for j in range(16):
                        bv = bb + cvs[j]
                        hv = plsc.load_gather(hbufs[a], [bv])
                        v = plsc.load_gather(row_bufs[a], [bv, hv + d0v])
                        plsc.store_scatter(out_bufs[a], [d0v, bv], v + posq[q])
                return carry

            lax.fori_loop(0, ngrp, b_body, 0)

        prep_idx(0, 0)
        start_gather(0)

        def step(s, a):
            b = 1 - a

            @pl.when(s >= 2)
            def _():
                pltpu.make_async_copy(out_bufs[a], out_dst(s - 2), sem_o[a]).wait()

            wait_gather(a)
            prep_idx(jnp.minimum(s + 1, seq - 1), b)
            start_gather(b)
            compute(s, a)
            pltpu.async_copy(out_bufs[a], out_dst(s), sem_o[a])

        def pair_body(s2, carry):
            step(2 * s2, 0)
            step(2 * s2 + 1, 1)
            return carry

        lax.fori_loop(0, seq // 2, pair_body, 0)
        wait_gather(0)
        pltpu.make_async_copy(out_bufs[0], out_dst(seq - 2), sem_o[0]).wait()
        pltpu.make_async_copy(out_bufs[1], out_dst(seq - 1), sem_o[1]).wait()

    return gather_kernel


def kernel(x, emb, posemb):
    batch, seq = x.shape
    voc, dim = emb.shape

    x_t = x.T
    emb_t = emb.T
    pos_t = posemb.T

    pairs = _make_transpose_kernel(voc, dim)(emb_t)
    ntail = (voc % 128) // 2
    if ntail:
        tail = emb[voc - 2 * ntail :, :].reshape(ntail, 2 * dim)
        pairs = lax.dynamic_update_slice(
            pairs, tail, (jnp.int32(voc // 2 - ntail), jnp.int32(0))
        )
    out_t = _make_gather_kernel(batch, seq, voc, dim)(x_t, pairs, pos_t)
    return out_t.transpose(2, 0, 1)


# speedup vs baseline: 2.5817x; 1.6882x over previous
import functools

import jax
import jax.numpy as jnp
from jax import lax
from jax.experimental import pallas as pl
from jax.experimental.pallas import tpu as pltpu
from jax.experimental.pallas import tpu_sc as plsc

LANES = 16
NUM_CORES = 2
NUM_SUBCORES = 16
NUM_WORKERS = NUM_CORES * NUM_SUBCORES

_MESH = dict(core_axis_name="c", subcore_axis_name="s", num_cores=NUM_CORES)


def _skew_vectors(dim):
    iota = lax.iota(jnp.int32, 16)
    iotav = tuple(iota + 16 * q for q in range(dim // 16))
    cvs = tuple(lax.rem(iota + j, 16) for j in range(16))
    return iotav, cvs


def _make_transpose_kernel(voc, dim):
    nblk_full = voc // 128
    iters = 246
    mesh = plsc.VectorSubcoreMesh(**_MESH)

    @functools.partial(
        pl.kernel,
        out_type=jax.ShapeDtypeStruct((voc, 2 * dim), jnp.float32),
        mesh=mesh,
        scratch_types=[
            pltpu.VMEM((dim, 128), jnp.float32),
            pltpu.VMEM((dim, 128), jnp.float32),
            pltpu.VMEM((128, 2 * dim), jnp.float32),
            pltpu.VMEM((128, 2 * dim), jnp.float32),
            pltpu.SemaphoreType.DMA,
            pltpu.SemaphoreType.DMA,
            pltpu.SemaphoreType.DMA,
            pltpu.SemaphoreType.DMA,
        ],
        compiler_params=pltpu.CompilerParams(use_tc_tiling_on_sc=True, needs_layout_passes=False),
    )
    def transpose_kernel(embt_hbm, table_hbm, in0, in1, out0, out1,
                         sem_i0, sem_i1, sem_s0, sem_s1):
        wid = lax.axis_index("s") * NUM_CORES + lax.axis_index("c")
        in_bufs, out_bufs = (in0, in1), (out0, out1)
        sem_i, sem_s = (sem_i0, sem_i1), (sem_s0, sem_s1)
        iotav, cvs = _skew_vectors(dim)

        def blk_id(k):
            return jnp.minimum(wid + NUM_WORKERS * k, nblk_full - 1)

        def in_src(k):
            return embt_hbm.at[pl.ds(0, dim), pl.ds(blk_id(k) * 128, 128)]

        def out_dst(k):
            return table_hbm.at[pl.ds(blk_id(k) * 128, 128)]

        def transpose_block(src, dst):
            def blk_body(t, carry):
                c0 = t * 16
                for q in range(dim // 16):
                    d0v = iotav[q]
                    for j in range(16):
                        c = cvs[j] + c0
                        v = plsc.load_gather(src, [d0v, c])
                        plsc.store_scatter(dst, [c, d0v], v)
                return carry

            lax.fori_loop(0, 8, blk_body, 0)

        pltpu.async_copy(in_src(0), in0, sem_i0)

        def step(k, a):
            b = 1 - a

            @pl.when(k >= 2)
            def _():
                pltpu.make_async_copy(out_bufs[a], out_dst(k - 2), sem_s[a]).wait()

            pltpu.make_async_copy(in_src(k), in_bufs[a], sem_i[a]).wait()
            pltpu.async_copy(in_src(k + 1), in_bufs[b], sem_i[b])
            transpose_block(in_bufs[a], out_bufs[a])
            pltpu.async_copy(out_bufs[a], out_dst(k), sem_s[a])

        def pair_body(k2, carry):
            step(2 * k2, 0)
            step(2 * k2 + 1, 1)
            return carry

        lax.fori_loop(0, iters // 2, pair_body, 0)
        pltpu.make_async_copy(in_src(iters), in_bufs[0], sem_i[0]).wait()
        pltpu.make_async_copy(out_bufs[0], out_dst(iters - 2), sem_s[0]).wait()
        pltpu.make_async_copy(out_bufs[1], out_dst(iters - 1), sem_s[1]).wait()

    return transpose_kernel


def _make_gather_kernel(batch, seq, voc, dim):
    b_per_w = batch // NUM_WORKERS
    mesh = plsc.VectorSubcoreMesh(**_MESH)

    @functools.partial(
        pl.kernel,
        out_type=jax.ShapeDtypeStruct((seq, dim, batch), jnp.float32),
        mesh=mesh,
        scratch_types=[
            pltpu.VMEM((seq, b_per_w), jnp.int32),
            pltpu.VMEM((dim, 256), jnp.float32),
            pltpu.VMEM((b_per_w, 2 * dim), jnp.float32),
            pltpu.VMEM((b_per_w, 2 * dim), jnp.float32),
            pltpu.VMEM((dim, b_per_w), jnp.float32),
            pltpu.VMEM((dim, b_per_w), jnp.float32),
            pltpu.SemaphoreType.DMA,
            pltpu.SemaphoreType.DMA,
            pltpu.SemaphoreType.DMA,
            pltpu.SemaphoreType.DMA,
        ],
        compiler_params=pltpu.CompilerParams(use_tc_tiling_on_sc=True, needs_layout_passes=False),
    )
    def gather_kernel(xt_hbm, table_hbm, pos_hbm, out_hbm,
                      x_v, pos_v, rows0, rows1, ov0, ov1,
                      sem_g0, sem_g1, sem_o0, sem_o1):
        wid = lax.axis_index("s") * NUM_CORES + lax.axis_index("c")
        b0 = wid * b_per_w
        row_bufs, out_bufs = (rows0, rows1), (ov0, ov1)
        sem_g, sem_o = (sem_g0, sem_g1), (sem_o0, sem_o1)
        ngrp = b_per_w // 16
        iotav, cvs = _skew_vectors(dim)

        pltpu.sync_copy(xt_hbm.at[pl.ds(0, seq), pl.ds(b0, b_per_w)], x_v)
        pltpu.sync_copy(pos_hbm.at[pl.ds(0, dim), pl.ds(0, 256)], pos_v)

        def start_gather(s, b):
            pltpu.async_copy(table_hbm.at[x_v.at[s]], row_bufs[b], sem_g[b])

        def wait_gather(s, b):
            pltpu.make_async_copy(
                table_hbm.at[x_v.at[s]], row_bufs[b], sem_g[b]
            ).wait()

        def out_dst(s):
            return out_hbm.at[s, pl.ds(0, dim), pl.ds(b0, b_per_w)]

        def compute(s, a):
            sv = jnp.full((16,), s, dtype=jnp.int32)
            posq = [
                plsc.load_gather(pos_v, [iotav[q], sv]) for q in range(dim // 16)
            ]

            def b_body(t, carry):
                bb = t * 16
                for q in range(dim // 16):
                    d0v = iotav[q]
                    for j in range(16):
                        bv = bb + cvs[j]
                        v = plsc.load_gather(row_bufs[a], [bv, d0v])
                        plsc.store_scatter(out_bufs[a], [d0v, bv], v + posq[q])
                return carry

            lax.fori_loop(0, ngrp, b_body, 0)

        start_gather(0, 0)

        def step(s, a):
            b = 1 - a

            @pl.when(s >= 2)
            def _():
                pltpu.make_async_copy(out_bufs[a], out_dst(s - 2), sem_o[a]).wait()

            wait_gather(s, a)
            start_gather(jnp.minimum(s + 1, seq - 1), b)
            compute(s, a)
            pltpu.async_copy(out_bufs[a], out_dst(s), sem_o[a])

        def pair_body(s2, carry):
            step(2 * s2, 0)
            step(2 * s2 + 1, 1)
            return carry

        lax.fori_loop(0, seq // 2, pair_body, 0)
        wait_gather(seq - 1, 0)
        pltpu.make_async_copy(out_bufs[0], out_dst(seq - 2), sem_o[0]).wait()
        pltpu.make_async_copy(out_bufs[1], out_dst(seq - 1), sem_o[1]).wait()

    return gather_kernel


def kernel(x, emb, posemb):
    batch, seq = x.shape
    voc, dim = emb.shape

    x_t = x.T
    emb_t = emb.T
    pos_t = posemb.T

    table = _make_transpose_kernel(voc, dim)(emb_t)
    ntail = voc % 128
    if ntail:
        tail = jnp.pad(emb[voc - ntail :, :], ((0, 0), (0, dim)))
        table = lax.dynamic_update_slice(
            table, tail, (jnp.int32(voc - ntail), jnp.int32(0))
        )
    out_t = _make_gather_kernel(batch, seq, voc, dim)(x_t, table, pos_t)
    return out_t.transpose(2, 0, 1)
